# baseline (speedup 1.0000x reference)
import functools

import jax
import jax.numpy as jnp
from jax import lax
from jax.experimental import pallas as pl
from jax.experimental.pallas import tpu as pltpu
from jax.experimental.pallas import tpu_sc as plsc

_N = 10000
_NP = 10240
_RPT = _NP // 16
_E = 320000
_CW = 128
_CHUNKS = 82
_TE = _CHUNKS * _CW
_EP = 32 * _TE


def _sc_mesh():
    return plsc.VectorSubcoreMesh(core_axis_name="c", subcore_axis_name="s")


def _sc_degree(dst_t, zeros1):
    @functools.partial(
        pl.kernel,
        out_type=jax.ShapeDtypeStruct((2, _NP), jnp.float32),
        mesh=_sc_mesh(),
        scratch_types=[
            pltpu.VMEM((_TE,), jnp.int32),
            pltpu.VMEM((_CW,), jnp.float32),
            pltpu.VMEM_SHARED((_NP,), jnp.float32),
        ],
    )
    def k(dst_hbm, z_hbm, out_hbm, idx_v, ones_v, acc):
        core = lax.axis_index("c")
        sid = lax.axis_index("s")
        g = core * 16 + sid
        for i in range(_CW // 16):
            ones_v[pl.ds(i * 16, 16)] = jnp.full((16,), 1.0, jnp.float32)
        pltpu.sync_copy(dst_hbm.at[g], idx_v)
        pltpu.sync_copy(z_hbm.at[pl.ds(sid * _RPT, _RPT)],
                        acc.at[pl.ds(sid * _RPT, _RPT)])
        plsc.subcore_barrier()

        def body(j, carry):
            pltpu.sync_copy(ones_v, acc.at[idx_v.at[pl.ds(j * _CW, _CW)]],
                            add=True)
            return carry

        lax.fori_loop(0, _CHUNKS, body, 0)
        plsc.subcore_barrier()
        pltpu.sync_copy(acc.at[pl.ds(sid * _RPT, _RPT)],
                        out_hbm.at[core, pl.ds(sid * _RPT, _RPT)])

    return k(dst_t, zeros1)


def _sc_scatter_rows(hp, src_t, dst_t, zeros_w, w):
    @functools.partial(
        pl.kernel,
        out_type=jax.ShapeDtypeStruct((2, _NP, w), jnp.float32),
        mesh=_sc_mesh(),
        scratch_types=[
            pltpu.VMEM((_TE,), jnp.int32),
            pltpu.VMEM((_TE,), jnp.int32),
            pltpu.VMEM((_CW, w), jnp.float32),
            pltpu.VMEM((_CW, w), jnp.float32),
            pltpu.VMEM_SHARED((_NP, w), jnp.float32),
            pltpu.SemaphoreType.DMA,
            pltpu.SemaphoreType.DMA,
        ],
        compiler_params=pltpu.CompilerParams(use_tc_tiling_on_sc=False),
    )
    def k(hp_hbm, src_hbm, dst_hbm, z_hbm, out_hbm,
          sidx, didx, rows0, rows1, acc, sem0, sem1):
        core = lax.axis_index("c")
        sid = lax.axis_index("s")
        g = core * 16 + sid
        pltpu.sync_copy(src_hbm.at[g], sidx)
        pltpu.sync_copy(dst_hbm.at[g], didx)
        pltpu.sync_copy(z_hbm.at[pl.ds(sid * _RPT, _RPT)],
                        acc.at[pl.ds(sid * _RPT, _RPT)])
        plsc.subcore_barrier()

        rows = (rows0, rows1)
        sems = (sem0, sem1)

        def gather(j, b):
            return pltpu.make_async_copy(
                hp_hbm.at[sidx.at[pl.ds(j * _CW, _CW)]], rows[b], sems[b])

        def scatter(j, b):
            pltpu.sync_copy(rows[b], acc.at[didx.at[pl.ds(j * _CW, _CW)]],
                            add=True)

        gather(0, 0).start()
        gather(1, 1).start()

        def body(jj, carry):
            j = jj * 2
            gather(j, 0).wait()
            scatter(j, 0)
            gather(j + 2, 0).start()
            gather(j + 1, 1).wait()
            scatter(j + 1, 1)
            gather(j + 3, 1).start()
            return carry

        lax.fori_loop(0, _CHUNKS // 2 - 1, body, 0)
        j = _CHUNKS - 2
        gather(j, 0).wait()
        scatter(j, 0)
        gather(j + 1, 1).wait()
        scatter(j + 1, 1)
        plsc.subcore_barrier()
        pltpu.sync_copy(acc.at[pl.ds(sid * _RPT, _RPT)],
                        out_hbm.at[core, pl.ds(sid * _RPT, _RPT)])

    return k(hp, src_t, dst_t, zeros_w)


def _tc_matmul1(x_pad, w1):
    def body(x_ref, w_ref, h_ref):
        h_ref[...] = jnp.dot(x_ref[...], w_ref[...],
                             preferred_element_type=jnp.float32)

    return pl.pallas_call(
        body,
        out_shape=jax.ShapeDtypeStruct((_NP, 64), jnp.float32),
    )(x_pad, w1)


def _tc_scale1(deg_parts, h):
    def body(dp_ref, h_ref, dinv_ref, hp_ref):
        deg = dp_ref[0] + dp_ref[1]
        dinv = jnp.where(deg > 0, lax.rsqrt(deg), 0.0)
        dinv_ref[...] = dinv
        hp_ref[...] = h_ref[...] * dinv

    return pl.pallas_call(
        body,
        out_shape=(jax.ShapeDtypeStruct((_NP, 1), jnp.float32),
                   jax.ShapeDtypeStruct((_NP, 64), jnp.float32)),
    )(deg_parts, h)


def _tc_dense2(s1_parts, dinv, w2):
    def body(sp_ref, dinv_ref, w_ref, out_ref):
        dinv = dinv_ref[...]
        h1 = jnp.maximum((sp_ref[0] + sp_ref[1]) * dinv, 0.0)
        out_ref[...] = jnp.dot(h1, w_ref[...],
                               preferred_element_type=jnp.float32) * dinv

    return pl.pallas_call(
        body,
        out_shape=jax.ShapeDtypeStruct((_NP, 16), jnp.float32),
    )(s1_parts, dinv, w2)


def _tc_z(s2_parts, dinv):
    def body(sp_ref, dinv_ref, z_ref):
        z_ref[...] = (sp_ref[0] + sp_ref[1]) * dinv_ref[...]

    return pl.pallas_call(
        body,
        out_shape=jax.ShapeDtypeStruct((_NP, 16), jnp.float32),
    )(s2_parts, dinv)


_RB = 200


def _tc_decoder(z, adj, pw, nrm):
    nr = _N // _RB

    def body(zi_ref, zj_ref, adj_ref, pw_ref, nrm_ref, out_ref):
        i = pl.program_id(0)
        logits = lax.dot_general(zi_ref[...].astype(jnp.bfloat16),
                                 zj_ref[...].astype(jnp.bfloat16),
                                 (((1,), (1,)), ((), ())),
                                 preferred_element_type=jnp.float32)
        a = adj_ref[...].astype(jnp.float32)
        pw = pw_ref[0, 0]
        log2e = 1.4426950408889634
        ln2 = 0.6931471805599453
        e = jnp.exp2(jnp.minimum(logits, -logits) * log2e)
        sp = jnp.maximum(logits, 0.0) + ln2 * jnp.log2(1.0 + e)
        le = sp + a * ((pw - 1.0) * sp - pw * logits)
        s = jnp.reshape(jnp.sum(le), (1, 1))

        @pl.when(i == 0)
        def _():
            out_ref[...] = jnp.zeros((1, 1), jnp.float32)

        out_ref[...] += s

        @pl.when(i == nr - 1)
        def _():
            out_ref[...] = out_ref[...] * (nrm_ref[0, 0] / (_N * _N))

    return pl.pallas_call(
        body,
        grid=(nr,),
        in_specs=[
            pl.BlockSpec((_RB, 16), lambda i: (i, 0)),
            pl.BlockSpec((_N, 16), lambda i: (0, 0)),
            pl.BlockSpec((_RB, _N), lambda i: (i, 0)),
            pl.BlockSpec(memory_space=pltpu.SMEM),
            pl.BlockSpec(memory_space=pltpu.SMEM),
        ],
        out_specs=pl.BlockSpec((1, 1), lambda i: (0, 0)),
        out_shape=jax.ShapeDtypeStruct((1, 1), jnp.float32),
    )(z, z, adj, pw, nrm)


def kernel(x, edge_index, adj_label, norm, W1, W2, pos_weight):
    loop = jnp.arange(_N, dtype=jnp.int32)
    pad = jnp.full((_EP - (_E + _N),), _N, jnp.int32)
    src = jnp.concatenate([edge_index[0], loop, pad]).reshape(32, _TE)
    dst = jnp.concatenate([edge_index[1], loop, pad]).reshape(32, _TE)
    x_pad = jnp.pad(x, ((0, _NP - _N), (0, 0)))
    zeros1 = jnp.zeros((_NP,), jnp.float32)
    zeros64 = jnp.zeros((_NP, 64), jnp.float32)
    zeros16 = jnp.zeros((_NP, 16), jnp.float32)

    deg_parts = _sc_degree(dst, zeros1)
    h = _tc_matmul1(x_pad, W1)
    dinv, hp1 = _tc_scale1(deg_parts.reshape(2, _NP, 1), h)
    s1 = _sc_scatter_rows(hp1, src, dst, zeros64, 64)
    hp2 = _tc_dense2(s1, dinv, W2)
    s2 = _sc_scatter_rows(hp2, src, dst, zeros16, 16)
    z_pad = _tc_z(s2, dinv)
    z = z_pad[:_N]
    cost = _tc_decoder(z, adj_label.astype(jnp.int8),
                       jnp.reshape(pos_weight, (1, 1)).astype(jnp.float32),
                       jnp.reshape(norm, (1, 1)).astype(jnp.float32))[0, 0]
    return (cost, z)


# speedup vs baseline: 1.2530x; 1.2530x over previous
import functools

import jax
import jax.numpy as jnp
from jax import lax
from jax.experimental import pallas as pl
from jax.experimental.pallas import tpu as pltpu
from jax.experimental.pallas import tpu_sc as plsc

_N = 10000
_NP = 10240
_RPT = _NP // 16
_E = 320000
_CW = 128
_CHUNKS = 82
_TE = _CHUNKS * _CW
_EP = 32 * _TE


def _sc_mesh():
    return plsc.VectorSubcoreMesh(core_axis_name="c", subcore_axis_name="s")


def _sc_degree(dst_t, zeros1):
    @functools.partial(
        pl.kernel,
        out_type=jax.ShapeDtypeStruct((2, _NP), jnp.float32),
        mesh=_sc_mesh(),
        scratch_types=[
            pltpu.VMEM((_TE,), jnp.int32),
            pltpu.VMEM((_CW,), jnp.float32),
            pltpu.VMEM_SHARED((_NP,), jnp.float32),
        ],
    )
    def k(dst_hbm, z_hbm, out_hbm, idx_v, ones_v, acc):
        core = lax.axis_index("c")
        sid = lax.axis_index("s")
        g = core * 16 + sid
        for i in range(_CW // 16):
            ones_v[pl.ds(i * 16, 16)] = jnp.full((16,), 1.0, jnp.float32)
        pltpu.sync_copy(dst_hbm.at[g], idx_v)
        pltpu.sync_copy(z_hbm.at[pl.ds(sid * _RPT, _RPT)],
                        acc.at[pl.ds(sid * _RPT, _RPT)])
        plsc.subcore_barrier()

        def body(j, carry):
            pltpu.sync_copy(ones_v, acc.at[idx_v.at[pl.ds(j * _CW, _CW)]],
                            add=True)
            return carry

        lax.fori_loop(0, _CHUNKS, body, 0)
        plsc.subcore_barrier()
        pltpu.sync_copy(acc.at[pl.ds(sid * _RPT, _RPT)],
                        out_hbm.at[core, pl.ds(sid * _RPT, _RPT)])

    return k(dst_t, zeros1)


def _sc_scatter_rows(hp, src_t, dst_t, zeros_w, w):
    @functools.partial(
        pl.kernel,
        out_type=jax.ShapeDtypeStruct((2, _NP, w), jnp.float32),
        mesh=_sc_mesh(),
        scratch_types=[
            pltpu.VMEM((_TE,), jnp.int32),
            pltpu.VMEM((_TE,), jnp.int32),
            pltpu.VMEM((_CW, w), jnp.float32),
            pltpu.VMEM((_CW, w), jnp.float32),
            pltpu.VMEM_SHARED((_NP, w), jnp.float32),
            pltpu.SemaphoreType.DMA,
            pltpu.SemaphoreType.DMA,
        ],
        compiler_params=pltpu.CompilerParams(use_tc_tiling_on_sc=False),
    )
    def k(hp_hbm, src_hbm, dst_hbm, z_hbm, out_hbm,
          sidx, didx, rows0, rows1, acc, sem0, sem1):
        core = lax.axis_index("c")
        sid = lax.axis_index("s")
        g = core * 16 + sid
        pltpu.sync_copy(src_hbm.at[g], sidx)
        pltpu.sync_copy(dst_hbm.at[g], didx)
        pltpu.sync_copy(z_hbm.at[pl.ds(sid * _RPT, _RPT)],
                        acc.at[pl.ds(sid * _RPT, _RPT)])
        plsc.subcore_barrier()

        rows = (rows0, rows1)
        sems = (sem0, sem1)

        def gather(j, b):
            return pltpu.make_async_copy(
                hp_hbm.at[sidx.at[pl.ds(j * _CW, _CW)]], rows[b], sems[b])

        def scatter(j, b):
            pltpu.sync_copy(rows[b], acc.at[didx.at[pl.ds(j * _CW, _CW)]],
                            add=True)

        gather(0, 0).start()
        gather(1, 1).start()

        def body(jj, carry):
            j = jj * 2
            gather(j, 0).wait()
            scatter(j, 0)
            gather(j + 2, 0).start()
            gather(j + 1, 1).wait()
            scatter(j + 1, 1)
            gather(j + 3, 1).start()
            return carry

        lax.fori_loop(0, _CHUNKS // 2 - 1, body, 0)
        j = _CHUNKS - 2
        gather(j, 0).wait()
        scatter(j, 0)
        gather(j + 1, 1).wait()
        scatter(j + 1, 1)
        plsc.subcore_barrier()
        pltpu.sync_copy(acc.at[pl.ds(sid * _RPT, _RPT)],
                        out_hbm.at[core, pl.ds(sid * _RPT, _RPT)])

    return k(hp, src_t, dst_t, zeros_w)


def _tc_matmul1(x_pad, w1):
    def body(x_ref, w_ref, h_ref):
        h_ref[...] = jnp.dot(x_ref[...], w_ref[...],
                             preferred_element_type=jnp.float32)

    return pl.pallas_call(
        body,
        out_shape=jax.ShapeDtypeStruct((_NP, 64), jnp.float32),
    )(x_pad, w1)


def _tc_scale1(deg_parts, h):
    def body(dp_ref, h_ref, dinv_ref, hp_ref):
        deg = dp_ref[0] + dp_ref[1]
        dinv = jnp.where(deg > 0, lax.rsqrt(deg), 0.0)
        dinv_ref[...] = dinv
        hp_ref[...] = h_ref[...] * dinv

    return pl.pallas_call(
        body,
        out_shape=(jax.ShapeDtypeStruct((_NP, 1), jnp.float32),
                   jax.ShapeDtypeStruct((_NP, 64), jnp.float32)),
    )(deg_parts, h)


def _tc_dense2(s1_parts, dinv, w2):
    def body(sp_ref, dinv_ref, w_ref, out_ref):
        dinv = dinv_ref[...]
        h1 = jnp.maximum((sp_ref[0] + sp_ref[1]) * dinv, 0.0)
        out_ref[...] = jnp.dot(h1, w_ref[...],
                               preferred_element_type=jnp.float32) * dinv

    return pl.pallas_call(
        body,
        out_shape=jax.ShapeDtypeStruct((_NP, 16), jnp.float32),
    )(s1_parts, dinv, w2)


def _tc_z(s2_parts, dinv):
    def body(sp_ref, dinv_ref, z_ref):
        z_ref[...] = (sp_ref[0] + sp_ref[1]) * dinv_ref[...]

    return pl.pallas_call(
        body,
        out_shape=jax.ShapeDtypeStruct((_NP, 16), jnp.float32),
    )(s2_parts, dinv)


_RB = 400


def _tc_decoder(z, adj, pw, nrm):
    nr = _N // _RB

    def body(zi_ref, zj_ref, adj_ref, pw_ref, nrm_ref, out_ref):
        i = pl.program_id(0)
        logits = lax.dot_general(zi_ref[...].astype(jnp.bfloat16),
                                 zj_ref[...].astype(jnp.bfloat16),
                                 (((1,), (1,)), ((), ())),
                                 preferred_element_type=jnp.float32)
        a = adj_ref[...]
        pw = pw_ref[0, 0]
        log2e = 1.4426950408889634
        ln2 = 0.6931471805599453
        e = jnp.exp2(jnp.minimum(logits, -logits) * log2e)
        sp = jnp.maximum(logits, 0.0) + ln2 * jnp.log2(1.0 + e)
        le = sp + a * ((pw - 1.0) * sp - pw * logits)
        s = jnp.reshape(jnp.sum(le), (1, 1))

        @pl.when(i == 0)
        def _():
            out_ref[...] = jnp.zeros((1, 1), jnp.float32)

        out_ref[...] += s

        @pl.when(i == nr - 1)
        def _():
            out_ref[...] = out_ref[...] * (nrm_ref[0, 0] / (_N * _N))

    return pl.pallas_call(
        body,
        grid=(nr,),
        in_specs=[
            pl.BlockSpec((_RB, 16), lambda i: (i, 0)),
            pl.BlockSpec((_N, 16), lambda i: (0, 0)),
            pl.BlockSpec((_RB, _N), lambda i: (i, 0)),
            pl.BlockSpec(memory_space=pltpu.SMEM),
            pl.BlockSpec(memory_space=pltpu.SMEM),
        ],
        out_specs=pl.BlockSpec((1, 1), lambda i: (0, 0)),
        out_shape=jax.ShapeDtypeStruct((1, 1), jnp.float32),
    )(z, z, adj, pw, nrm)


def kernel(x, edge_index, adj_label, norm, W1, W2, pos_weight):
    loop = jnp.arange(_N, dtype=jnp.int32)
    pad = jnp.full((_EP - (_E + _N),), _N, jnp.int32)
    src = jnp.concatenate([edge_index[0], loop, pad]).reshape(32, _TE)
    dst = jnp.concatenate([edge_index[1], loop, pad]).reshape(32, _TE)
    x_pad = jnp.pad(x, ((0, _NP - _N), (0, 0)))
    zeros1 = jnp.zeros((_NP,), jnp.float32)
    zeros64 = jnp.zeros((_NP, 64), jnp.float32)
    zeros16 = jnp.zeros((_NP, 16), jnp.float32)

    deg_parts = _sc_degree(dst, zeros1)
    h = _tc_matmul1(x_pad, W1)
    dinv, hp1 = _tc_scale1(deg_parts.reshape(2, _NP, 1), h)
    s1 = _sc_scatter_rows(hp1, src, dst, zeros64, 64)
    hp2 = _tc_dense2(s1, dinv, W2)
    s2 = _sc_scatter_rows(hp2, src, dst, zeros16, 16)
    z_pad = _tc_z(s2, dinv)
    z = z_pad[:_N]
    cost = _tc_decoder(z, adj_label,
                       jnp.reshape(pos_weight, (1, 1)).astype(jnp.float32),
                       jnp.reshape(norm, (1, 1)).astype(jnp.float32))[0, 0]
    return (cost, z)


# speedup vs baseline: 1.2541x; 1.0009x over previous
import functools

import jax
import jax.numpy as jnp
from jax import lax
from jax.experimental import pallas as pl
from jax.experimental.pallas import tpu as pltpu
from jax.experimental.pallas import tpu_sc as plsc

_N = 10000
_NP = 10240
_RPT = _NP // 16
_E = 320000
_CW = 128
_CHUNKS = 82
_TE = _CHUNKS * _CW
_EP = 32 * _TE


def _sc_mesh():
    return plsc.VectorSubcoreMesh(core_axis_name="c", subcore_axis_name="s")


def _sc_degree(dst_t, zeros1):
    @functools.partial(
        pl.kernel,
        out_type=jax.ShapeDtypeStruct((2, _NP), jnp.float32),
        mesh=_sc_mesh(),
        scratch_types=[
            pltpu.VMEM((_TE,), jnp.int32),
            pltpu.VMEM((_CW,), jnp.float32),
            pltpu.VMEM_SHARED((_NP,), jnp.float32),
        ],
    )
    def k(dst_hbm, z_hbm, out_hbm, idx_v, ones_v, acc):
        core = lax.axis_index("c")
        sid = lax.axis_index("s")
        g = core * 16 + sid
        for i in range(_CW // 16):
            ones_v[pl.ds(i * 16, 16)] = jnp.full((16,), 1.0, jnp.float32)
        pltpu.sync_copy(dst_hbm.at[g], idx_v)
        pltpu.sync_copy(z_hbm.at[pl.ds(sid * _RPT, _RPT)],
                        acc.at[pl.ds(sid * _RPT, _RPT)])
        plsc.subcore_barrier()

        def body(j, carry):
            pltpu.sync_copy(ones_v, acc.at[idx_v.at[pl.ds(j * _CW, _CW)]],
                            add=True)
            return carry

        lax.fori_loop(0, _CHUNKS, body, 0)
        plsc.subcore_barrier()
        pltpu.sync_copy(acc.at[pl.ds(sid * _RPT, _RPT)],
                        out_hbm.at[core, pl.ds(sid * _RPT, _RPT)])

    return k(dst_t, zeros1)


def _sc_scatter_rows(hp, src_t, dst_t, zeros_w, w):
    @functools.partial(
        pl.kernel,
        out_type=jax.ShapeDtypeStruct((2, _NP, w), jnp.float32),
        mesh=_sc_mesh(),
        scratch_types=[
            pltpu.VMEM((_TE,), jnp.int32),
            pltpu.VMEM((_TE,), jnp.int32),
        ] + [pltpu.VMEM((_CW, w), jnp.float32) for _ in range(6)] + [
            pltpu.VMEM_SHARED((_NP, w), jnp.float32),
        ] + [pltpu.SemaphoreType.DMA for _ in range(12)],
        compiler_params=pltpu.CompilerParams(use_tc_tiling_on_sc=False),
    )
    def k(hp_hbm, src_hbm, dst_hbm, z_hbm, out_hbm,
          sidx, didx, r0, r1, r2, r3, r4, r5, acc,
          g0, g1, g2, g3, g4, g5, s0, s1, s2, s3, s4, s5):
        core = lax.axis_index("c")
        sid = lax.axis_index("s")
        g = core * 16 + sid
        pltpu.sync_copy(src_hbm.at[g], sidx)
        pltpu.sync_copy(dst_hbm.at[g], didx)
        pltpu.sync_copy(z_hbm.at[pl.ds(sid * _RPT, _RPT)],
                        acc.at[pl.ds(sid * _RPT, _RPT)])
        plsc.subcore_barrier()

        rows = (r0, r1, r2, r3, r4, r5)
        gsems = (g0, g1, g2, g3, g4, g5)
        ssems = (s0, s1, s2, s3, s4, s5)

        def gstart(j, b):
            d = pltpu.make_async_copy(
                hp_hbm.at[sidx.at[pl.ds(j * _CW, _CW)]], rows[b], gsems[b])
            d.start()
            return d

        def sstart(j, b):
            return pltpu.async_copy(
                rows[b], acc.at[didx.at[pl.ds(j * _CW, _CW)]], ssems[b],
                add=True)

        ng = _CHUNKS // 2
        gd, sd = {}, {}
        for b in range(4):
            gd[b] = gstart(b, b)
        for i in range(ng):
            b0 = 2 * (i % 3)
            gd[2 * i].wait()
            sd[2 * i] = sstart(2 * i, b0)
            gd[2 * i + 1].wait()
            sd[2 * i + 1] = sstart(2 * i + 1, b0 + 1)
            if i >= 1:
                sd[2 * i - 2].wait()
                sd[2 * i - 1].wait()
            if i + 2 < ng:
                q = 2 * ((i + 2) % 3)
                gd[2 * i + 4] = gstart(2 * i + 4, q)
                gd[2 * i + 5] = gstart(2 * i + 5, q + 1)
        sd[_CHUNKS - 2].wait()
        sd[_CHUNKS - 1].wait()
        plsc.subcore_barrier()
        pltpu.sync_copy(acc.at[pl.ds(sid * _RPT, _RPT)],
                        out_hbm.at[core, pl.ds(sid * _RPT, _RPT)])

    return k(hp, src_t, dst_t, zeros_w)


def _tc_matmul1(x_pad, w1):
    def body(x_ref, w_ref, h_ref):
        h_ref[...] = jnp.dot(x_ref[...], w_ref[...],
                             preferred_element_type=jnp.float32)

    return pl.pallas_call(
        body,
        out_shape=jax.ShapeDtypeStruct((_NP, 64), jnp.float32),
    )(x_pad, w1)


def _tc_scale1(deg_parts, h):
    def body(dp_ref, h_ref, dinv_ref, hp_ref):
        deg = dp_ref[0] + dp_ref[1]
        dinv = jnp.where(deg > 0, lax.rsqrt(deg), 0.0)
        dinv_ref[...] = dinv
        hp_ref[...] = h_ref[...] * dinv

    return pl.pallas_call(
        body,
        out_shape=(jax.ShapeDtypeStruct((_NP, 1), jnp.float32),
                   jax.ShapeDtypeStruct((_NP, 64), jnp.float32)),
    )(deg_parts, h)


def _tc_dense2(s1_parts, dinv, w2):
    def body(sp_ref, dinv_ref, w_ref, out_ref):
        dinv = dinv_ref[...]
        h1 = jnp.maximum((sp_ref[0] + sp_ref[1]) * dinv, 0.0)
        out_ref[...] = jnp.dot(h1, w_ref[...],
                               preferred_element_type=jnp.float32) * dinv

    return pl.pallas_call(
        body,
        out_shape=jax.ShapeDtypeStruct((_NP, 16), jnp.float32),
    )(s1_parts, dinv, w2)


def _tc_z(s2_parts, dinv):
    def body(sp_ref, dinv_ref, z_ref):
        z_ref[...] = (sp_ref[0] + sp_ref[1]) * dinv_ref[...]

    return pl.pallas_call(
        body,
        out_shape=jax.ShapeDtypeStruct((_NP, 16), jnp.float32),
    )(s2_parts, dinv)


_RB = 400


def _tc_decoder(z, adj, pw, nrm):
    nr = _N // _RB

    def body(zi_ref, zj_ref, adj_ref, pw_ref, nrm_ref, out_ref):
        i = pl.program_id(0)
        logits = lax.dot_general(zi_ref[...].astype(jnp.bfloat16),
                                 zj_ref[...].astype(jnp.bfloat16),
                                 (((1,), (1,)), ((), ())),
                                 preferred_element_type=jnp.float32)
        a = adj_ref[...]
        pw = pw_ref[0, 0]
        log2e = 1.4426950408889634
        ln2 = 0.6931471805599453
        e = jnp.exp2(jnp.minimum(logits, -logits) * log2e)
        sp = jnp.maximum(logits, 0.0) + ln2 * jnp.log2(1.0 + e)
        le = sp + a * ((pw - 1.0) * sp - pw * logits)
        s = jnp.reshape(jnp.sum(le), (1, 1))

        @pl.when(i == 0)
        def _():
            out_ref[...] = jnp.zeros((1, 1), jnp.float32)

        out_ref[...] += s

        @pl.when(i == nr - 1)
        def _():
            out_ref[...] = out_ref[...] * (nrm_ref[0, 0] / (_N * _N))

    return pl.pallas_call(
        body,
        grid=(nr,),
        in_specs=[
            pl.BlockSpec((_RB, 16), lambda i: (i, 0)),
            pl.BlockSpec((_N, 16), lambda i: (0, 0)),
            pl.BlockSpec((_RB, _N), lambda i: (i, 0)),
            pl.BlockSpec(memory_space=pltpu.SMEM),
            pl.BlockSpec(memory_space=pltpu.SMEM),
        ],
        out_specs=pl.BlockSpec((1, 1), lambda i: (0, 0)),
        out_shape=jax.ShapeDtypeStruct((1, 1), jnp.float32),
    )(z, z, adj, pw, nrm)


def kernel(x, edge_index, adj_label, norm, W1, W2, pos_weight):
    loop = jnp.arange(_N, dtype=jnp.int32)
    pad = jnp.full((_EP - (_E + _N),), _N, jnp.int32)
    src = jnp.concatenate([edge_index[0], loop, pad]).reshape(32, _TE)
    dst = jnp.concatenate([edge_index[1], loop, pad]).reshape(32, _TE)
    x_pad = jnp.pad(x, ((0, _NP - _N), (0, 0)))
    zeros1 = jnp.zeros((_NP,), jnp.float32)
    zeros64 = jnp.zeros((_NP, 64), jnp.float32)
    zeros16 = jnp.zeros((_NP, 16), jnp.float32)

    deg_parts = _sc_degree(dst, zeros1)
    h = _tc_matmul1(x_pad, W1)
    dinv, hp1 = _tc_scale1(deg_parts.reshape(2, _NP, 1), h)
    s1 = _sc_scatter_rows(hp1, src, dst, zeros64, 64)
    hp2 = _tc_dense2(s1, dinv, W2)
    s2 = _sc_scatter_rows(hp2, src, dst, zeros16, 16)
    z_pad = _tc_z(s2, dinv)
    z = z_pad[:_N]
    cost = _tc_decoder(z, adj_label,
                       jnp.reshape(pos_weight, (1, 1)).astype(jnp.float32),
                       jnp.reshape(norm, (1, 1)).astype(jnp.float32))[0, 0]
    return (cost, z)


# speedup vs baseline: 1.3408x; 1.0691x over previous
import functools

import jax
import jax.numpy as jnp
from jax import lax
from jax.experimental import pallas as pl
from jax.experimental.pallas import tpu as pltpu
from jax.experimental.pallas import tpu_sc as plsc

_N = 10000
_NP = 10240
_RPT = _NP // 16
_E = 320000
_CW = 128
_CHUNKS = 82
_TE = _CHUNKS * _CW
_EP = 32 * _TE


def _sc_mesh():
    return plsc.VectorSubcoreMesh(core_axis_name="c", subcore_axis_name="s")


def _sc_degree(dst_t, zeros1):
    @functools.partial(
        pl.kernel,
        out_type=jax.ShapeDtypeStruct((2, _NP), jnp.float32),
        mesh=_sc_mesh(),
        scratch_types=[
            pltpu.VMEM((_TE,), jnp.int32),
            pltpu.VMEM((_CW,), jnp.float32),
            pltpu.VMEM_SHARED((_NP,), jnp.float32),
        ],
    )
    def k(dst_hbm, z_hbm, out_hbm, idx_v, ones_v, acc):
        core = lax.axis_index("c")
        sid = lax.axis_index("s")
        g = core * 16 + sid
        for i in range(_CW // 16):
            ones_v[pl.ds(i * 16, 16)] = jnp.full((16,), 1.0, jnp.float32)
        pltpu.sync_copy(dst_hbm.at[g], idx_v)
        pltpu.sync_copy(z_hbm.at[pl.ds(sid * _RPT, _RPT)],
                        acc.at[pl.ds(sid * _RPT, _RPT)])
        plsc.subcore_barrier()

        def body(j, carry):
            pltpu.sync_copy(ones_v, acc.at[idx_v.at[pl.ds(j * _CW, _CW)]],
                            add=True)
            return carry

        lax.fori_loop(0, _CHUNKS, body, 0)
        plsc.subcore_barrier()
        pltpu.sync_copy(acc.at[pl.ds(sid * _RPT, _RPT)],
                        out_hbm.at[core, pl.ds(sid * _RPT, _RPT)])

    return k(dst_t, zeros1)


def _sc_scatter_rows(hp, src_t, dst_t, zeros_w, w):
    @functools.partial(
        pl.kernel,
        out_type=jax.ShapeDtypeStruct((2, _NP, w), jnp.float32),
        mesh=_sc_mesh(),
        scratch_types=[
            pltpu.VMEM((_TE,), jnp.int32),
            pltpu.VMEM((_TE,), jnp.int32),
        ] + [pltpu.VMEM((_CW, w), jnp.float32) for _ in range(6)] + [
            pltpu.VMEM_SHARED((_NP, w), jnp.float32),
        ] + [pltpu.SemaphoreType.DMA for _ in range(12)],
        compiler_params=pltpu.CompilerParams(use_tc_tiling_on_sc=False),
    )
    def k(hp_hbm, src_hbm, dst_hbm, z_hbm, out_hbm,
          sidx, didx, r0, r1, r2, r3, r4, r5, acc,
          g0, g1, g2, g3, g4, g5, s0, s1, s2, s3, s4, s5):
        core = lax.axis_index("c")
        sid = lax.axis_index("s")
        g = core * 16 + sid
        pltpu.sync_copy(src_hbm.at[g], sidx)
        pltpu.sync_copy(dst_hbm.at[g], didx)
        pltpu.sync_copy(z_hbm.at[pl.ds(sid * _RPT, _RPT)],
                        acc.at[pl.ds(sid * _RPT, _RPT)])
        plsc.subcore_barrier()

        rows = (r0, r1, r2, r3, r4, r5)
        gsems = (g0, g1, g2, g3, g4, g5)
        ssems = (s0, s1, s2, s3, s4, s5)

        def gstart(j, b):
            d = pltpu.make_async_copy(
                hp_hbm.at[sidx.at[pl.ds(j * _CW, _CW)]], rows[b], gsems[b])
            d.start()
            return d

        def sstart(j, b):
            return pltpu.async_copy(
                rows[b], acc.at[didx.at[pl.ds(j * _CW, _CW)]], ssems[b],
                add=True)

        ng = _CHUNKS // 2
        gd, sd = {}, {}
        for b in range(4):
            gd[b] = gstart(b, b)
        for i in range(ng):
            b0 = 2 * (i % 3)
            gd[2 * i].wait()
            sd[2 * i] = sstart(2 * i, b0)
            gd[2 * i + 1].wait()
            sd[2 * i + 1] = sstart(2 * i + 1, b0 + 1)
            if i >= 1:
                sd[2 * i - 2].wait()
                sd[2 * i - 1].wait()
            if i + 2 < ng:
                q = 2 * ((i + 2) % 3)
                gd[2 * i + 4] = gstart(2 * i + 4, q)
                gd[2 * i + 5] = gstart(2 * i + 5, q + 1)
        sd[_CHUNKS - 2].wait()
        sd[_CHUNKS - 1].wait()
        plsc.subcore_barrier()
        pltpu.sync_copy(acc.at[pl.ds(sid * _RPT, _RPT)],
                        out_hbm.at[core, pl.ds(sid * _RPT, _RPT)])

    return k(hp, src_t, dst_t, zeros_w)


def _tc_matmul1(x_pad, w1):
    def body(x_ref, w_ref, h_ref):
        h_ref[...] = jnp.dot(x_ref[...], w_ref[...],
                             preferred_element_type=jnp.float32)

    return pl.pallas_call(
        body,
        out_shape=jax.ShapeDtypeStruct((_NP, 64), jnp.float32),
    )(x_pad, w1)


def _tc_scale1(deg_parts, h):
    def body(dp_ref, h_ref, dinv_ref, hp_ref):
        deg = dp_ref[0] + dp_ref[1]
        dinv = jnp.where(deg > 0, lax.rsqrt(deg), 0.0)
        dinv_ref[...] = dinv
        hp_ref[...] = h_ref[...] * dinv

    return pl.pallas_call(
        body,
        out_shape=(jax.ShapeDtypeStruct((_NP, 1), jnp.float32),
                   jax.ShapeDtypeStruct((_NP, 64), jnp.float32)),
    )(deg_parts, h)


def _tc_dense2(s1_parts, dinv, w2):
    def body(sp_ref, dinv_ref, w_ref, out_ref):
        dinv = dinv_ref[...]
        h1 = jnp.maximum((sp_ref[0] + sp_ref[1]) * dinv, 0.0)
        out_ref[...] = jnp.dot(h1, w_ref[...],
                               preferred_element_type=jnp.float32) * dinv

    return pl.pallas_call(
        body,
        out_shape=jax.ShapeDtypeStruct((_NP, 16), jnp.float32),
    )(s1_parts, dinv, w2)


_LOG2E = 1.4426950408889634
_LN2 = 0.6931471805599453


def _tc_z(s2_parts, dinv):
    def body(sp_ref, dinv_ref, z_ref, zs_ref):
        z = (sp_ref[0] + sp_ref[1]) * dinv_ref[...]
        z_ref[...] = z
        zs_ref[...] = z * _LOG2E

    return pl.pallas_call(
        body,
        out_shape=(jax.ShapeDtypeStruct((_NP, 16), jnp.float32),
                   jax.ShapeDtypeStruct((_NP, 16), jnp.float32)),
    )(s2_parts, dinv)


_RB = 400


def _tc_decoder(zs, z, adj, pw, nrm):
    nr = _N // _RB

    def body(zsi_ref, zj_ref, adj_ref, pw_ref, nrm_ref, out_ref):
        i = pl.program_id(0)
        zsi = zsi_ref[...]
        zjb = zj_ref[...].astype(jnp.bfloat16)
        ls = lax.dot_general(zsi.astype(jnp.bfloat16), zjb,
                             (((1,), (1,)), ((), ())),
                             preferred_element_type=jnp.float32)
        a = adj_ref[...]
        pw = pw_ref[0, 0]
        e = jnp.exp2(jnp.minimum(ls, -ls))
        spb = jnp.maximum(ls, 0.0) + jnp.log2(1.0 + e)
        az = lax.dot_general(a.astype(jnp.bfloat16), zjb,
                             (((1,), (0,)), ((), ())),
                             preferred_element_type=jnp.float32)
        s = (jnp.sum(spb) + (pw - 1.0) * jnp.sum(a * spb)
             - pw * jnp.sum(zsi * az))
        s = jnp.reshape(s, (1, 1))

        @pl.when(i == 0)
        def _():
            out_ref[...] = jnp.zeros((1, 1), jnp.float32)

        out_ref[...] += s

        @pl.when(i == nr - 1)
        def _():
            out_ref[...] = out_ref[...] * (nrm_ref[0, 0] * (_LN2 / (_N * _N)))

    return pl.pallas_call(
        body,
        grid=(nr,),
        in_specs=[
            pl.BlockSpec((_RB, 16), lambda i: (i, 0)),
            pl.BlockSpec((_N, 16), lambda i: (0, 0)),
            pl.BlockSpec((_RB, _N), lambda i: (i, 0)),
            pl.BlockSpec(memory_space=pltpu.SMEM),
            pl.BlockSpec(memory_space=pltpu.SMEM),
        ],
        out_specs=pl.BlockSpec((1, 1), lambda i: (0, 0)),
        out_shape=jax.ShapeDtypeStruct((1, 1), jnp.float32),
    )(zs, z, adj, pw, nrm)


def kernel(x, edge_index, adj_label, norm, W1, W2, pos_weight):
    loop = jnp.arange(_N, dtype=jnp.int32)
    pad = jnp.full((_EP - (_E + _N),), _N, jnp.int32)
    src = jnp.concatenate([edge_index[0], loop, pad]).reshape(32, _TE)
    dst = jnp.concatenate([edge_index[1], loop, pad]).reshape(32, _TE)
    x_pad = jnp.pad(x, ((0, _NP - _N), (0, 0)))
    zeros1 = jnp.zeros((_NP,), jnp.float32)
    zeros64 = jnp.zeros((_NP, 64), jnp.float32)
    zeros16 = jnp.zeros((_NP, 16), jnp.float32)

    deg_parts = _sc_degree(dst, zeros1)
    h = _tc_matmul1(x_pad, W1)
    dinv, hp1 = _tc_scale1(deg_parts.reshape(2, _NP, 1), h)
    s1 = _sc_scatter_rows(hp1, src, dst, zeros64, 64)
    hp2 = _tc_dense2(s1, dinv, W2)
    s2 = _sc_scatter_rows(hp2, src, dst, zeros16, 16)
    z_pad, zs_pad = _tc_z(s2, dinv)
    z = z_pad[:_N]
    cost = _tc_decoder(zs_pad[:_N], z, adj_label,
                       jnp.reshape(pos_weight, (1, 1)).astype(jnp.float32),
                       jnp.reshape(norm, (1, 1)).astype(jnp.float32))[0, 0]
    return (cost, z)
